# baseline (speedup 1.0000x reference)
import functools

import jax
import jax.numpy as jnp
from jax import lax
from jax.experimental import pallas as pl
from jax.experimental.pallas import tpu as pltpu
from jax.experimental.pallas import tpu_sc as plsc

N = 10000
D = 128
E = 320000
MIN_NORM = 1e-15
EPS = 1e-5
K_FRAC = 0.75

NC = 2
NS = 16
NW = NC * NS
L = 16

NP = 10240
NPW = NP // NS
EPW = 10240
EP = NW * EPW
NY2 = 10240
LPW = D // NW
CSZ = 4096
NCH = EP // CSZ

_mesh = plsc.VectorSubcoreMesh(
    core_axis_name="c", subcore_axis_name="s", num_cores=NC, num_subcores=NS)


def _tc1_body(x_ref, w_ref, xt_ref, s_ref, a1_ref, a2_ref):
    xv = x_ref[...]
    nrm = jnp.sqrt(jnp.sum(xv * xv, axis=1, keepdims=True))
    nrm = jnp.maximum(nrm, MIN_NORM)
    yc = jnp.clip(nrm, -1.0 + 1e-7, 1.0 - 1e-7)
    at = 0.5 * (jnp.log1p(yc) - jnp.log1p(-yc))
    xt = xv / nrm * at
    xt_ref[...] = xt
    s_ref[...] = jnp.sum(xv, axis=1, keepdims=True)
    w1 = w_ref[0:1, 0:D]
    w2 = w_ref[0:1, D:2 * D]
    a1_ref[...] = jnp.sum(xt * w1, axis=1, keepdims=True)
    a2_ref[...] = jnp.sum(xt * w2, axis=1, keepdims=True)


_tc1 = pl.pallas_call(
    _tc1_body,
    out_shape=[
        jax.ShapeDtypeStruct((N, D), jnp.float32),
        jax.ShapeDtypeStruct((N, 1), jnp.float32),
        jax.ShapeDtypeStruct((N, 1), jnp.float32),
        jax.ShapeDtypeStruct((N, 1), jnp.float32),
    ],
)


def _tc2_body(d0_ref, d1_ref, s_ref, dis_ref, p_ref):
    deg = d0_ref[...] + d1_ref[...]
    dis = jnp.where(deg > 0, lax.rsqrt(deg), 0.0)
    dis_ref[...] = dis
    p_ref[...] = dis * s_ref[...]


_tc2 = pl.pallas_call(
    _tc2_body,
    out_shape=[
        jax.ShapeDtypeStruct((N, 1), jnp.float32),
        jax.ShapeDtypeStruct((N, 1), jnp.float32),
    ],
)


def _tc3_body(q0_ref, q1_ref, s_ref, dis_ref, a1_ref, a2_ref,
              sel_ref, g_ref):
    score = s_ref[...] + dis_ref[...] * (q0_ref[...] + q1_ref[...])
    kth = int(N * K_FRAC)
    key = lax.bitcast_convert_type(score, jnp.int32)
    key = jnp.where(key >= 0, key, key ^ jnp.int32(0x7FFFFFFF))
    u = lax.bitcast_convert_type(key, jnp.uint32) ^ jnp.uint32(0x80000000)
    t = jnp.uint32(0)
    for bit in range(31, -1, -1):
        cand = t | (jnp.uint32(1) << jnp.uint32(bit))
        cnt = jnp.sum((u >= cand).astype(jnp.int32))
        t = jnp.where(cnt >= kth, cand, t)
    sel = (u > t).astype(jnp.float32)
    sel_ref[...] = sel
    g_ref[...] = sel * a1_ref[...] + a2_ref[...]


_tc3 = pl.pallas_call(
    _tc3_body,
    out_shape=[
        jax.ShapeDtypeStruct((N, 1), jnp.float32),
        jax.ShapeDtypeStruct((N, 1), jnp.float32),
    ],
)


def _tc4_body(w0_ref, w1_ref, sel_ref, b_ref, xt_ref, y_ref):
    z = w0_ref[...] + w1_ref[...] + b_ref[0, 0]
    weight = 1.0 / (1.0 + jnp.exp(-z))
    coef = weight * sel_ref[...]
    y_ref[...] = coef * xt_ref[...]


_tc4 = pl.pallas_call(
    _tc4_body,
    out_shape=jax.ShapeDtypeStruct((N, D), jnp.float32),
)


def _tc5_body(xt_ref, a_ref, out_ref):
    ot = xt_ref[...] + jnp.maximum(a_ref[...], 0.0)
    un = jnp.sqrt(jnp.sum(ot * ot, axis=1, keepdims=True))
    un = jnp.maximum(un, MIN_NORM)
    ex = jnp.tanh(un) * ot / un
    en = jnp.sqrt(jnp.sum(ex * ex, axis=1, keepdims=True))
    en = jnp.maximum(en, MIN_NORM)
    maxnorm = 1.0 - EPS
    out_ref[...] = jnp.where(en > maxnorm, ex / en * maxnorm, ex)


_tc5 = pl.pallas_call(
    _tc5_body,
    out_shape=jax.ShapeDtypeStruct((N, D), jnp.float32),
)


@functools.partial(
    pl.kernel,
    out_type=jax.ShapeDtypeStruct((NC, NP), jnp.float32),
    mesh=_mesh,
    compiler_params=pltpu.CompilerParams(needs_layout_passes=False),
    scratch_types=[
        pltpu.VMEM((EPW,), jnp.int32),
        pltpu.VMEM((EPW,), jnp.int32),
        pltpu.VMEM((NP,), jnp.float32),
        pltpu.VMEM((NP,), jnp.float32),
        pltpu.VMEM((NS, NPW), jnp.float32),
        pltpu.VMEM((NPW,), jnp.float32),
        pltpu.VMEM_SHARED((NS, NP), jnp.float32),
        pltpu.SemaphoreType.DMA,
    ],
)
def _sc_scalar_seg(gidx_hbm, sidx_hbm, val_hbm, out_hbm,
                   gidx_v, sidx_v, val_v, acc_v, red_v, out_v, stage_sh, sem):
    cid = lax.axis_index("c")
    sid = lax.axis_index("s")
    wid = sid * NC + cid
    pltpu.sync_copy(gidx_hbm.at[wid], gidx_v)
    pltpu.sync_copy(sidx_hbm.at[wid], sidx_v)
    pltpu.sync_copy(val_hbm, val_v)

    @plsc.parallel_loop(0, NP // L, unroll=8)
    def zero_body(i):
        acc_v[pl.ds(i * L, L)] = jnp.zeros((L,), jnp.float32)

    @plsc.parallel_loop(0, EPW // L, unroll=8)
    def edge_body(i):
        gi = gidx_v[pl.ds(i * L, L)]
        si = sidx_v[pl.ds(i * L, L)]
        v = plsc.load_gather(val_v, [gi])
        plsc.addupdate_scatter(acc_v, [si], v)

    pltpu.sync_copy(acc_v, stage_sh.at[sid])
    plsc.subcore_barrier()
    pltpu.sync_copy(stage_sh.at[:, pl.ds(sid * NPW, NPW)], red_v)

    @plsc.parallel_loop(0, NPW // L, unroll=4)
    def red_body(i):
        sl = pl.ds(i * L, L)
        acc16 = red_v[0, sl]
        for r in range(1, NS):
            acc16 = acc16 + red_v[r, sl]
        out_v[sl] = acc16
    pltpu.sync_copy(out_v, out_hbm.at[cid, pl.ds(sid * NPW, NPW)])


@functools.partial(
    pl.kernel,
    out_type=jax.ShapeDtypeStruct((NW, LPW, NY2), jnp.float32),
    mesh=_mesh,
    compiler_params=pltpu.CompilerParams(needs_layout_passes=False),
    scratch_types=[
        pltpu.VMEM((LPW, NY2), jnp.float32),
        pltpu.VMEM((LPW, NY2), jnp.float32),
        pltpu.VMEM((CSZ,), jnp.int32),
        pltpu.VMEM((CSZ,), jnp.int32),
        pltpu.SemaphoreType.DMA,
        pltpu.SemaphoreType.DMA,
    ],
)
def _sc_vec_seg(pe_hbm, yt_hbm, out_hbm, yv, accv, pe0, pe1, sem0, sem1):
    cid = lax.axis_index("c")
    sid = lax.axis_index("s")
    wid = sid * NC + cid
    pltpu.sync_copy(yt_hbm.at[wid], yv)

    @plsc.parallel_loop(0, NY2 // L, unroll=8)
    def zb(i):
        for l in range(LPW):
            accv[l, pl.ds(i * L, L)] = jnp.zeros((L,), jnp.float32)

    lane_idx = [jnp.full((L,), l, jnp.int32) for l in range(LPW)]

    def _process(buf):
        @plsc.parallel_loop(0, CSZ // L, unroll=16)
        def gb(g):
            pk = buf[pl.ds(g * L, L)]
            row16 = jnp.bitwise_and(pk, jnp.int32(16383))
            col16 = lax.shift_right_logical(pk, 14)
            for l in range(LPW):
                v = plsc.load_gather(yv, [lane_idx[l], row16])
                plsc.addupdate_scatter(accv, [lane_idx[l], col16], v)

    pltpu.sync_copy(pe_hbm.at[0], pe0)
    pltpu.async_copy(pe_hbm.at[1], pe1, sem1)

    def chunk_body(m, c):
        j = 2 * m
        more = m + 1 < NCH // 2

        @pl.when(m > 0)
        def _():
            pltpu.make_async_copy(pe_hbm.at[j], pe0, sem0).wait()
        _process(pe0)

        @pl.when(more)
        def _():
            pltpu.async_copy(pe_hbm.at[j + 2], pe0, sem0)
        pltpu.make_async_copy(pe_hbm.at[j + 1], pe1, sem1).wait()
        _process(pe1)

        @pl.when(more)
        def _():
            pltpu.async_copy(pe_hbm.at[j + 3], pe1, sem1)
        return c
    lax.fori_loop(0, NCH // 2, chunk_body, 0)

    pltpu.sync_copy(accv, out_hbm.at[wid])


def kernel(x, edge_index, W, b):
    row = edge_index[0]
    col = edge_index[1]
    pad = EP - E
    rowp = jnp.concatenate([row, jnp.full((pad,), N, jnp.int32)])
    colp = jnp.concatenate([col, jnp.full((pad,), N, jnp.int32)])
    rowf = rowp.reshape(NW, EPW)
    colf = colp.reshape(NW, EPW)
    packed = jnp.bitwise_or(rowp, colp << 14).reshape(NCH, CSZ)

    xt, s, a1, a2 = _tc1(x, W)

    ones_np = jnp.ones((NP,), jnp.float32)
    degp = _sc_scalar_seg(rowf, rowf, ones_np)
    dis, p = _tc2(degp[0, :N, None], degp[1, :N, None], s)

    pf = jnp.pad(p[:, 0], (0, NP - N))
    qp = _sc_scalar_seg(rowf, colf, pf)
    sel, g = _tc3(qp[0, :N, None], qp[1, :N, None], s, dis, a1, a2)

    gf = jnp.pad(g[:, 0], (0, NP - N))
    wp = _sc_scalar_seg(rowf, colf, gf)
    y = _tc4(wp[0, :N, None], wp[1, :N, None], sel, b.reshape(1, 1), xt)

    ypad = jnp.concatenate([y, jnp.zeros((NY2 - N, D), jnp.float32)], axis=0)
    yt3 = ypad.T.reshape(NW, LPW, NY2)
    at3 = _sc_vec_seg(packed, yt3)
    a_full = at3.reshape(D, NY2).T
    out = _tc5(xt, a_full[:N])
    return out


# speedup vs baseline: 1.1689x; 1.1689x over previous
import functools

import jax
import jax.numpy as jnp
from jax import lax
from jax.experimental import pallas as pl
from jax.experimental.pallas import tpu as pltpu
from jax.experimental.pallas import tpu_sc as plsc

N = 10000
D = 128
E = 320000
MIN_NORM = 1e-15
EPS = 1e-5
K_FRAC = 0.75

NC = 2
NS = 16
NW = NC * NS
L = 16

NP = 10240
NPW = NP // NS
EPW = 10240
EP = NW * EPW
NY2 = 10240
LPW = D // NW
CSZ = 4096
NCH = EP // CSZ

_mesh = plsc.VectorSubcoreMesh(
    core_axis_name="c", subcore_axis_name="s", num_cores=NC, num_subcores=NS)


def _tc1_body(x_ref, w_ref, xt_ref, s_ref, a1_ref, a2_ref):
    xv = x_ref[...]
    nrm = jnp.sqrt(jnp.sum(xv * xv, axis=1, keepdims=True))
    nrm = jnp.maximum(nrm, MIN_NORM)
    yc = jnp.clip(nrm, -1.0 + 1e-7, 1.0 - 1e-7)
    at = 0.5 * (jnp.log1p(yc) - jnp.log1p(-yc))
    xt = xv / nrm * at
    xt_ref[...] = xt
    s_ref[...] = jnp.sum(xv, axis=1, keepdims=True)
    w1 = w_ref[0:1, 0:D]
    w2 = w_ref[0:1, D:2 * D]
    a1_ref[...] = jnp.sum(xt * w1, axis=1, keepdims=True)
    a2_ref[...] = jnp.sum(xt * w2, axis=1, keepdims=True)


_tc1 = pl.pallas_call(
    _tc1_body,
    out_shape=[
        jax.ShapeDtypeStruct((N, D), jnp.float32),
        jax.ShapeDtypeStruct((N, 1), jnp.float32),
        jax.ShapeDtypeStruct((N, 1), jnp.float32),
        jax.ShapeDtypeStruct((N, 1), jnp.float32),
    ],
)


def _tc2_body(d0_ref, d1_ref, s_ref, dis_ref, p_ref):
    deg = d0_ref[...] + d1_ref[...]
    dis = jnp.where(deg > 0, lax.rsqrt(deg), 0.0)
    dis_ref[...] = dis
    p_ref[...] = dis * s_ref[...]


_tc2 = pl.pallas_call(
    _tc2_body,
    out_shape=[
        jax.ShapeDtypeStruct((N, 1), jnp.float32),
        jax.ShapeDtypeStruct((N, 1), jnp.float32),
    ],
)


def _tc3_body(q0_ref, q1_ref, s_ref, dis_ref, a1_ref, a2_ref,
              sel_ref, g_ref):
    score = s_ref[...] + dis_ref[...] * (q0_ref[...] + q1_ref[...])
    kth = int(N * K_FRAC)
    key = lax.bitcast_convert_type(score, jnp.int32)
    key = jnp.where(key >= 0, key, key ^ jnp.int32(0x7FFFFFFF))
    u = lax.bitcast_convert_type(key, jnp.uint32) ^ jnp.uint32(0x80000000)
    t = jnp.uint32(0)
    for bit in range(31, -1, -1):
        cand = t | (jnp.uint32(1) << jnp.uint32(bit))
        cnt = jnp.sum((u >= cand).astype(jnp.int32))
        t = jnp.where(cnt >= kth, cand, t)
    sel = (u > t).astype(jnp.float32)
    sel_ref[...] = sel
    g_ref[...] = sel * a1_ref[...] + a2_ref[...]


_tc3 = pl.pallas_call(
    _tc3_body,
    out_shape=[
        jax.ShapeDtypeStruct((N, 1), jnp.float32),
        jax.ShapeDtypeStruct((N, 1), jnp.float32),
    ],
)


def _tc4_body(w0_ref, w1_ref, sel_ref, b_ref, xt_ref, y_ref):
    z = w0_ref[...] + w1_ref[...] + b_ref[0, 0]
    weight = 1.0 / (1.0 + jnp.exp(-z))
    coef = weight * sel_ref[...]
    y_ref[...] = coef * xt_ref[...]


_tc4 = pl.pallas_call(
    _tc4_body,
    out_shape=jax.ShapeDtypeStruct((N, D), jnp.float32),
)


def _tc5_body(xt_ref, a_ref, out_ref):
    ot = xt_ref[...] + jnp.maximum(a_ref[...], 0.0)
    un = jnp.sqrt(jnp.sum(ot * ot, axis=1, keepdims=True))
    un = jnp.maximum(un, MIN_NORM)
    ex = jnp.tanh(un) * ot / un
    en = jnp.sqrt(jnp.sum(ex * ex, axis=1, keepdims=True))
    en = jnp.maximum(en, MIN_NORM)
    maxnorm = 1.0 - EPS
    out_ref[...] = jnp.where(en > maxnorm, ex / en * maxnorm, ex)


_tc5 = pl.pallas_call(
    _tc5_body,
    out_shape=jax.ShapeDtypeStruct((N, D), jnp.float32),
)


@functools.partial(
    pl.kernel,
    out_type=jax.ShapeDtypeStruct((NC, NP), jnp.float32),
    mesh=_mesh,
    compiler_params=pltpu.CompilerParams(needs_layout_passes=False),
    scratch_types=[
        pltpu.VMEM((EPW,), jnp.int32),
        pltpu.VMEM((EPW,), jnp.int32),
        pltpu.VMEM((NP,), jnp.float32),
        pltpu.VMEM((NP,), jnp.float32),
        pltpu.VMEM((NS, NPW), jnp.float32),
        pltpu.VMEM((NPW,), jnp.float32),
        pltpu.VMEM_SHARED((NS, NP), jnp.float32),
        pltpu.SemaphoreType.DMA,
    ],
)
def _sc_scalar_seg(gidx_hbm, sidx_hbm, val_hbm, out_hbm,
                   gidx_v, sidx_v, val_v, acc_v, red_v, out_v, stage_sh, sem):
    cid = lax.axis_index("c")
    sid = lax.axis_index("s")
    wid = sid * NC + cid
    pltpu.sync_copy(gidx_hbm.at[wid], gidx_v)
    pltpu.sync_copy(sidx_hbm.at[wid], sidx_v)
    pltpu.sync_copy(val_hbm, val_v)

    @plsc.parallel_loop(0, NP // L, unroll=8)
    def zero_body(i):
        acc_v[pl.ds(i * L, L)] = jnp.zeros((L,), jnp.float32)

    @plsc.parallel_loop(0, EPW // L, unroll=8)
    def edge_body(i):
        gi = gidx_v[pl.ds(i * L, L)]
        si = sidx_v[pl.ds(i * L, L)]
        v = plsc.load_gather(val_v, [gi])
        plsc.addupdate_scatter(acc_v, [si], v)

    pltpu.sync_copy(acc_v, stage_sh.at[sid])
    plsc.subcore_barrier()
    pltpu.sync_copy(stage_sh.at[:, pl.ds(sid * NPW, NPW)], red_v)

    @plsc.parallel_loop(0, NPW // L, unroll=4)
    def red_body(i):
        sl = pl.ds(i * L, L)
        acc16 = red_v[0, sl]
        for r in range(1, NS):
            acc16 = acc16 + red_v[r, sl]
        out_v[sl] = acc16
    pltpu.sync_copy(out_v, out_hbm.at[cid, pl.ds(sid * NPW, NPW)])


@functools.partial(
    pl.kernel,
    out_type=jax.ShapeDtypeStruct((NW, LPW, NY2), jnp.float32),
    mesh=_mesh,
    compiler_params=pltpu.CompilerParams(needs_layout_passes=False),
    scratch_types=[
        pltpu.VMEM((LPW, NY2), jnp.float32),
        pltpu.VMEM((LPW, NY2), jnp.float32),
        pltpu.VMEM((CSZ,), jnp.int32),
        pltpu.VMEM((CSZ,), jnp.int32),
        pltpu.SemaphoreType.DMA,
        pltpu.SemaphoreType.DMA,
    ],
)
def _sc_vec_seg(pe_hbm, yt_hbm, out_hbm, yv, accv, pe0, pe1, sem0, sem1):
    cid = lax.axis_index("c")
    sid = lax.axis_index("s")
    wid = sid * NC + cid
    pltpu.sync_copy(yt_hbm.at[wid], yv)

    @plsc.parallel_loop(0, NY2 // L, unroll=8)
    def zb(i):
        for l in range(LPW):
            accv[l, pl.ds(i * L, L)] = jnp.zeros((L,), jnp.float32)

    lane_idx = [jnp.full((L,), l, jnp.int32) for l in range(LPW)]

    def _process(buf):
        @plsc.parallel_loop(0, CSZ // L, unroll=8)
        def gb(g):
            pk = buf[pl.ds(g * L, L)]
            row16 = jnp.bitwise_and(pk, jnp.int32(16383))
            col16 = lax.shift_right_logical(pk, 14)
            for l in range(LPW):
                v = plsc.load_gather(yv, [lane_idx[l], row16])
                plsc.addupdate_scatter(accv, [lane_idx[l], col16], v)

    pltpu.sync_copy(pe_hbm.at[0], pe0)
    pltpu.async_copy(pe_hbm.at[1], pe1, sem1)

    def chunk_body(m, c):
        j = 2 * m
        more = m + 1 < NCH // 2

        @pl.when(m > 0)
        def _():
            pltpu.make_async_copy(pe_hbm.at[j], pe0, sem0).wait()
        _process(pe0)

        @pl.when(more)
        def _():
            pltpu.async_copy(pe_hbm.at[j + 2], pe0, sem0)
        pltpu.make_async_copy(pe_hbm.at[j + 1], pe1, sem1).wait()
        _process(pe1)

        @pl.when(more)
        def _():
            pltpu.async_copy(pe_hbm.at[j + 3], pe1, sem1)
        return c
    lax.fori_loop(0, NCH // 2, chunk_body, 0)

    pltpu.sync_copy(accv, out_hbm.at[wid])


def kernel(x, edge_index, W, b):
    row = edge_index[0]
    col = edge_index[1]
    pad = EP - E
    rowp = jnp.concatenate([row, jnp.full((pad,), N, jnp.int32)])
    colp = jnp.concatenate([col, jnp.full((pad,), N, jnp.int32)])
    rowf = rowp.reshape(NW, EPW)
    colf = colp.reshape(NW, EPW)
    packed = jnp.bitwise_or(rowp, colp << 14).reshape(NCH, CSZ)

    xt, s, a1, a2 = _tc1(x, W)

    ones_np = jnp.ones((NP,), jnp.float32)
    degp = _sc_scalar_seg(rowf, rowf, ones_np)
    dis, p = _tc2(degp[0, :N, None], degp[1, :N, None], s)

    pf = jnp.pad(p[:, 0], (0, NP - N))
    qp = _sc_scalar_seg(rowf, colf, pf)
    sel, g = _tc3(qp[0, :N, None], qp[1, :N, None], s, dis, a1, a2)

    gf = jnp.pad(g[:, 0], (0, NP - N))
    wp = _sc_scalar_seg(rowf, colf, gf)
    y = _tc4(wp[0, :N, None], wp[1, :N, None], sel, b.reshape(1, 1), xt)

    ypad = jnp.concatenate([y, jnp.zeros((NY2 - N, D), jnp.float32)], axis=0)
    yt3 = ypad.T.reshape(NW, LPW, NY2)
    at3 = _sc_vec_seg(packed, yt3)
    a_full = at3.reshape(D, NY2).T
    out = _tc5(xt, a_full[:N])
    return out


# speedup vs baseline: 1.1717x; 1.0024x over previous
import functools

import jax
import jax.numpy as jnp
from jax import lax
from jax.experimental import pallas as pl
from jax.experimental.pallas import tpu as pltpu
from jax.experimental.pallas import tpu_sc as plsc

N = 10000
D = 128
E = 320000
MIN_NORM = 1e-15
EPS = 1e-5
K_FRAC = 0.75

NC = 2
NS = 16
NW = NC * NS
L = 16

NP = 10240
NPW = NP // NS
EPW = 10240
EP = NW * EPW
NY2 = 10240
LPW = D // NW
CSZ = 4096
NCH = EP // CSZ

_mesh = plsc.VectorSubcoreMesh(
    core_axis_name="c", subcore_axis_name="s", num_cores=NC, num_subcores=NS)


def _tc1_body(x_ref, w_ref, xt_ref, s_ref, a1_ref, a2_ref):
    xv = x_ref[...]
    nrm = jnp.sqrt(jnp.sum(xv * xv, axis=1, keepdims=True))
    nrm = jnp.maximum(nrm, MIN_NORM)
    yc = jnp.clip(nrm, -1.0 + 1e-7, 1.0 - 1e-7)
    at = 0.5 * (jnp.log1p(yc) - jnp.log1p(-yc))
    xt = xv / nrm * at
    xt_ref[...] = xt
    s_ref[...] = jnp.sum(xv, axis=1, keepdims=True)
    w1 = w_ref[0:1, 0:D]
    w2 = w_ref[0:1, D:2 * D]
    a1_ref[...] = jnp.sum(xt * w1, axis=1, keepdims=True)
    a2_ref[...] = jnp.sum(xt * w2, axis=1, keepdims=True)


_tc1 = pl.pallas_call(
    _tc1_body,
    out_shape=[
        jax.ShapeDtypeStruct((N, D), jnp.float32),
        jax.ShapeDtypeStruct((N, 1), jnp.float32),
        jax.ShapeDtypeStruct((N, 1), jnp.float32),
        jax.ShapeDtypeStruct((N, 1), jnp.float32),
    ],
)


def _tc2_body(d0_ref, d1_ref, s_ref, dis_ref, p_ref):
    deg = d0_ref[...] + d1_ref[...]
    dis = jnp.where(deg > 0, lax.rsqrt(deg), 0.0)
    dis_ref[...] = dis
    p_ref[...] = dis * s_ref[...]


_tc2 = pl.pallas_call(
    _tc2_body,
    out_shape=[
        jax.ShapeDtypeStruct((N, 1), jnp.float32),
        jax.ShapeDtypeStruct((N, 1), jnp.float32),
    ],
)


def _tc3_body(q0_ref, q1_ref, s_ref, dis_ref, a1_ref, a2_ref,
              sel_ref, g_ref):
    score = s_ref[...] + dis_ref[...] * (q0_ref[...] + q1_ref[...])
    kth = int(N * K_FRAC)
    key = lax.bitcast_convert_type(score, jnp.int32)
    key = jnp.where(key >= 0, key, key ^ jnp.int32(0x7FFFFFFF))
    u = lax.bitcast_convert_type(key, jnp.uint32) ^ jnp.uint32(0x80000000)
    t = jnp.uint32(0)
    for bit in range(31, -1, -1):
        cand = t | (jnp.uint32(1) << jnp.uint32(bit))
        cnt = jnp.sum((u >= cand).astype(jnp.int32))
        t = jnp.where(cnt >= kth, cand, t)
    sel = (u > t).astype(jnp.float32)
    sel_ref[...] = sel
    g_ref[...] = sel * a1_ref[...] + a2_ref[...]


_tc3 = pl.pallas_call(
    _tc3_body,
    out_shape=[
        jax.ShapeDtypeStruct((N, 1), jnp.float32),
        jax.ShapeDtypeStruct((N, 1), jnp.float32),
    ],
)


def _tc4_body(w0_ref, w1_ref, sel_ref, b_ref, coef_ref):
    z = w0_ref[...] + w1_ref[...] + b_ref[0, 0]
    weight = 1.0 / (1.0 + jnp.exp(-z))
    coef_ref[...] = weight * sel_ref[...]


_tc4 = pl.pallas_call(
    _tc4_body,
    out_shape=jax.ShapeDtypeStruct((N, 1), jnp.float32),
)


def _tc5_body(xt_ref, a_ref, out_ref):
    ot = xt_ref[...] + jnp.maximum(a_ref[...], 0.0)
    un = jnp.sqrt(jnp.sum(ot * ot, axis=1, keepdims=True))
    un = jnp.maximum(un, MIN_NORM)
    ex = jnp.tanh(un) * ot / un
    en = jnp.sqrt(jnp.sum(ex * ex, axis=1, keepdims=True))
    en = jnp.maximum(en, MIN_NORM)
    maxnorm = 1.0 - EPS
    out_ref[...] = jnp.where(en > maxnorm, ex / en * maxnorm, ex)


_tc5 = pl.pallas_call(
    _tc5_body,
    out_shape=jax.ShapeDtypeStruct((N, D), jnp.float32),
)


@functools.partial(
    pl.kernel,
    out_type=jax.ShapeDtypeStruct((NC, NP), jnp.float32),
    mesh=_mesh,
    compiler_params=pltpu.CompilerParams(needs_layout_passes=False),
    scratch_types=[
        pltpu.VMEM((EPW,), jnp.int32),
        pltpu.VMEM((EPW,), jnp.int32),
        pltpu.VMEM((NP,), jnp.float32),
        pltpu.VMEM((NP,), jnp.float32),
        pltpu.VMEM((NS, NPW), jnp.float32),
        pltpu.VMEM((NPW,), jnp.float32),
        pltpu.VMEM_SHARED((NS, NP), jnp.float32),
        pltpu.SemaphoreType.DMA,
    ],
)
def _sc_scalar_seg(gidx_hbm, sidx_hbm, val_hbm, out_hbm,
                   gidx_v, sidx_v, val_v, acc_v, red_v, out_v, stage_sh, sem):
    cid = lax.axis_index("c")
    sid = lax.axis_index("s")
    wid = sid * NC + cid
    pltpu.sync_copy(gidx_hbm.at[wid], gidx_v)
    pltpu.sync_copy(sidx_hbm.at[wid], sidx_v)
    pltpu.sync_copy(val_hbm, val_v)

    @plsc.parallel_loop(0, NP // L, unroll=8)
    def zero_body(i):
        acc_v[pl.ds(i * L, L)] = jnp.zeros((L,), jnp.float32)

    @plsc.parallel_loop(0, EPW // L, unroll=8)
    def edge_body(i):
        gi = gidx_v[pl.ds(i * L, L)]
        si = sidx_v[pl.ds(i * L, L)]
        v = plsc.load_gather(val_v, [gi])
        plsc.addupdate_scatter(acc_v, [si], v)

    pltpu.sync_copy(acc_v, stage_sh.at[sid])
    plsc.subcore_barrier()
    pltpu.sync_copy(stage_sh.at[:, pl.ds(sid * NPW, NPW)], red_v)

    @plsc.parallel_loop(0, NPW // L, unroll=4)
    def red_body(i):
        sl = pl.ds(i * L, L)
        acc16 = red_v[0, sl]
        for r in range(1, NS):
            acc16 = acc16 + red_v[r, sl]
        out_v[sl] = acc16
    pltpu.sync_copy(out_v, out_hbm.at[cid, pl.ds(sid * NPW, NPW)])


@functools.partial(
    pl.kernel,
    out_type=jax.ShapeDtypeStruct((NW, LPW, NY2), jnp.float32),
    mesh=_mesh,
    compiler_params=pltpu.CompilerParams(needs_layout_passes=False),
    scratch_types=[
        pltpu.VMEM((LPW, NY2), jnp.float32),
        pltpu.VMEM((LPW, NY2), jnp.float32),
        pltpu.VMEM((NY2,), jnp.float32),
        pltpu.VMEM((CSZ,), jnp.int32),
        pltpu.VMEM((CSZ,), jnp.int32),
        pltpu.SemaphoreType.DMA,
        pltpu.SemaphoreType.DMA,
    ],
)
def _sc_vec_seg(pe_hbm, yt_hbm, coef_hbm, out_hbm,
                yv, accv, coef_v, pe0, pe1, sem0, sem1):
    cid = lax.axis_index("c")
    sid = lax.axis_index("s")
    wid = sid * NC + cid
    pltpu.sync_copy(yt_hbm.at[wid], yv)
    pltpu.sync_copy(coef_hbm, coef_v)

    @plsc.parallel_loop(0, NY2 // L, unroll=8)
    def zb(i):
        sl = pl.ds(i * L, L)
        c16 = coef_v[sl]
        for l in range(LPW):
            accv[l, sl] = jnp.zeros((L,), jnp.float32)
            yv[l, sl] = yv[l, sl] * c16

    lane_idx = [jnp.full((L,), l, jnp.int32) for l in range(LPW)]

    def _process(buf):
        @plsc.parallel_loop(0, CSZ // L, unroll=8)
        def gb(g):
            pk = buf[pl.ds(g * L, L)]
            row16 = jnp.bitwise_and(pk, jnp.int32(16383))
            col16 = lax.shift_right_logical(pk, 14)
            for l in range(LPW):
                v = plsc.load_gather(yv, [lane_idx[l], row16])
                plsc.addupdate_scatter(accv, [lane_idx[l], col16], v)

    pltpu.sync_copy(pe_hbm.at[0], pe0)
    pltpu.async_copy(pe_hbm.at[1], pe1, sem1)

    def chunk_body(m, c):
        j = 2 * m
        more = m + 1 < NCH // 2

        @pl.when(m > 0)
        def _():
            pltpu.make_async_copy(pe_hbm.at[j], pe0, sem0).wait()
        _process(pe0)

        @pl.when(more)
        def _():
            pltpu.async_copy(pe_hbm.at[j + 2], pe0, sem0)
        pltpu.make_async_copy(pe_hbm.at[j + 1], pe1, sem1).wait()
        _process(pe1)

        @pl.when(more)
        def _():
            pltpu.async_copy(pe_hbm.at[j + 3], pe1, sem1)
        return c
    lax.fori_loop(0, NCH // 2, chunk_body, 0)

    pltpu.sync_copy(accv, out_hbm.at[wid])


def kernel(x, edge_index, W, b):
    row = edge_index[0]
    col = edge_index[1]
    pad = EP - E
    rowp = jnp.concatenate([row, jnp.full((pad,), N, jnp.int32)])
    colp = jnp.concatenate([col, jnp.full((pad,), N, jnp.int32)])
    rowf = rowp.reshape(NW, EPW)
    colf = colp.reshape(NW, EPW)
    packed = jnp.bitwise_or(rowp, colp << 14).reshape(NCH, CSZ)

    xt, s, a1, a2 = _tc1(x, W)

    ones_np = jnp.ones((NP,), jnp.float32)
    degp = _sc_scalar_seg(rowf, rowf, ones_np)
    dis, p = _tc2(degp[0, :N, None], degp[1, :N, None], s)

    pf = jnp.pad(p[:, 0], (0, NP - N))
    qp = _sc_scalar_seg(rowf, colf, pf)
    sel, g = _tc3(qp[0, :N, None], qp[1, :N, None], s, dis, a1, a2)

    gf = jnp.pad(g[:, 0], (0, NP - N))
    wp = _sc_scalar_seg(rowf, colf, gf)
    coef = _tc4(wp[0, :N, None], wp[1, :N, None], sel, b.reshape(1, 1))
    coefp = jnp.pad(coef[:, 0], (0, NY2 - N))

    xtpad = jnp.concatenate([xt, jnp.zeros((NY2 - N, D), jnp.float32)], axis=0)
    xt3 = xtpad.T.reshape(NW, LPW, NY2)
    at3 = _sc_vec_seg(packed, xt3, coefp)
    a_full = at3.reshape(D, NY2).T
    out = _tc5(xt, a_full[:N])
    return out


# speedup vs baseline: 1.2354x; 1.0544x over previous
import functools

import jax
import jax.numpy as jnp
from jax import lax
from jax.experimental import pallas as pl
from jax.experimental.pallas import tpu as pltpu
from jax.experimental.pallas import tpu_sc as plsc

N = 10000
D = 128
E = 320000
MIN_NORM = 1e-15
EPS = 1e-5
K_FRAC = 0.75

NC = 2
NS = 16
NW = NC * NS
L = 16

NP = 10240
NPW = NP // NS
EPW = 10240
EP = NW * EPW
NY2 = 10240
LPW = D // NW
CSZ = 4096
NCH = EP // CSZ

_mesh = plsc.VectorSubcoreMesh(
    core_axis_name="c", subcore_axis_name="s", num_cores=NC, num_subcores=NS)


def _tc1_body(x_ref, w_ref, xt_ref, s_ref, a1_ref, a2_ref):
    xv = x_ref[...]
    nrm = jnp.sqrt(jnp.sum(xv * xv, axis=1, keepdims=True))
    nrm = jnp.maximum(nrm, MIN_NORM)
    yc = jnp.clip(nrm, -1.0 + 1e-7, 1.0 - 1e-7)
    at = 0.5 * (jnp.log1p(yc) - jnp.log1p(-yc))
    xt = xv / nrm * at
    xt_ref[...] = xt
    s_ref[...] = jnp.sum(xv, axis=1, keepdims=True)
    w1 = w_ref[0:1, 0:D]
    w2 = w_ref[0:1, D:2 * D]
    a1_ref[...] = jnp.sum(xt * w1, axis=1, keepdims=True)
    a2_ref[...] = jnp.sum(xt * w2, axis=1, keepdims=True)


_tc1 = pl.pallas_call(
    _tc1_body,
    out_shape=[
        jax.ShapeDtypeStruct((N, D), jnp.float32),
        jax.ShapeDtypeStruct((N, 1), jnp.float32),
        jax.ShapeDtypeStruct((N, 1), jnp.float32),
        jax.ShapeDtypeStruct((N, 1), jnp.float32),
    ],
)


def _tc3_body(q0_ref, q1_ref, d0_ref, d1_ref, s_ref, a1_ref, a2_ref,
              sel_ref, g_ref):
    deg = d0_ref[...] + d1_ref[...]
    dis = jnp.where(deg > 0, lax.rsqrt(deg), 0.0)
    score = s_ref[...] + dis * (q0_ref[...] + q1_ref[...])
    kth = int(N * K_FRAC)
    key = lax.bitcast_convert_type(score, jnp.int32)
    key = jnp.where(key >= 0, key, key ^ jnp.int32(0x7FFFFFFF))
    u = lax.bitcast_convert_type(key, jnp.uint32) ^ jnp.uint32(0x80000000)
    t = jnp.uint32(0)
    for bit in range(31, -1, -1):
        cand = t | (jnp.uint32(1) << jnp.uint32(bit))
        cnt = jnp.sum((u >= cand).astype(jnp.int32))
        t = jnp.where(cnt >= kth, cand, t)
    sel = (u > t).astype(jnp.float32)
    sel_ref[...] = sel
    g_ref[...] = sel * a1_ref[...] + a2_ref[...]


_tc3 = pl.pallas_call(
    _tc3_body,
    out_shape=[
        jax.ShapeDtypeStruct((N, 1), jnp.float32),
        jax.ShapeDtypeStruct((N, 1), jnp.float32),
    ],
)


def _tc5_body(xt_ref, a_ref, out_ref):
    ot = xt_ref[...] + jnp.maximum(a_ref[...], 0.0)
    un = jnp.sqrt(jnp.sum(ot * ot, axis=1, keepdims=True))
    un = jnp.maximum(un, MIN_NORM)
    ex = jnp.tanh(un) * ot / un
    en = jnp.sqrt(jnp.sum(ex * ex, axis=1, keepdims=True))
    en = jnp.maximum(en, MIN_NORM)
    maxnorm = 1.0 - EPS
    out_ref[...] = jnp.where(en > maxnorm, ex / en * maxnorm, ex)


_tc5 = pl.pallas_call(
    _tc5_body,
    out_shape=jax.ShapeDtypeStruct((N, D), jnp.float32),
)


@functools.partial(
    pl.kernel,
    out_type=jax.ShapeDtypeStruct((NC, NP), jnp.float32),
    mesh=_mesh,
    compiler_params=pltpu.CompilerParams(needs_layout_passes=False),
    scratch_types=[
        pltpu.VMEM((EPW,), jnp.int32),
        pltpu.VMEM((EPW,), jnp.int32),
        pltpu.VMEM((NP,), jnp.float32),
        pltpu.VMEM((NP,), jnp.float32),
        pltpu.VMEM((NS, NPW), jnp.float32),
        pltpu.VMEM((NPW,), jnp.float32),
        pltpu.VMEM_SHARED((NS, NP), jnp.float32),
        pltpu.SemaphoreType.DMA,
    ],
)
def _sc_scalar_seg(gidx_hbm, sidx_hbm, val_hbm, out_hbm,
                   gidx_v, sidx_v, val_v, acc_v, red_v, out_v, stage_sh, sem):
    cid = lax.axis_index("c")
    sid = lax.axis_index("s")
    wid = sid * NC + cid
    pltpu.sync_copy(gidx_hbm.at[wid], gidx_v)
    pltpu.sync_copy(sidx_hbm.at[wid], sidx_v)
    pltpu.sync_copy(val_hbm, val_v)

    @plsc.parallel_loop(0, NP // L, unroll=8)
    def zero_body(i):
        acc_v[pl.ds(i * L, L)] = jnp.zeros((L,), jnp.float32)

    @plsc.parallel_loop(0, EPW // L, unroll=8)
    def edge_body(i):
        gi = gidx_v[pl.ds(i * L, L)]
        si = sidx_v[pl.ds(i * L, L)]
        v = plsc.load_gather(val_v, [gi])
        plsc.addupdate_scatter(acc_v, [si], v)

    pltpu.sync_copy(acc_v, stage_sh.at[sid])
    plsc.subcore_barrier()
    pltpu.sync_copy(stage_sh.at[:, pl.ds(sid * NPW, NPW)], red_v)

    @plsc.parallel_loop(0, NPW // L, unroll=4)
    def red_body(i):
        sl = pl.ds(i * L, L)
        acc16 = red_v[0, sl]
        for r in range(1, NS):
            acc16 = acc16 + red_v[r, sl]
        out_v[sl] = acc16
    pltpu.sync_copy(out_v, out_hbm.at[cid, pl.ds(sid * NPW, NPW)])


@functools.partial(
    pl.kernel,
    out_type=jax.ShapeDtypeStruct((NC, NP), jnp.float32),
    mesh=_mesh,
    compiler_params=pltpu.CompilerParams(needs_layout_passes=False),
    scratch_types=[
        pltpu.VMEM((EPW,), jnp.int32),
        pltpu.VMEM((EPW,), jnp.int32),
        pltpu.VMEM((NP,), jnp.float32),
        pltpu.VMEM((NP,), jnp.float32),
        pltpu.VMEM((NP,), jnp.float32),
        pltpu.VMEM((NP,), jnp.float32),
        pltpu.VMEM((NS, NPW), jnp.float32),
        pltpu.VMEM((NPW,), jnp.float32),
        pltpu.VMEM_SHARED((NS, NP), jnp.float32),
        pltpu.SemaphoreType.DMA,
    ],
)
def _sc_score_seg(gidx_hbm, sidx_hbm, deg_hbm, s_hbm, out_hbm,
                  gidx_v, sidx_v, d0_v, d1_v, val_v, acc_v, red_v, out_v,
                  stage_sh, sem):
    cid = lax.axis_index("c")
    sid = lax.axis_index("s")
    wid = sid * NC + cid
    pltpu.sync_copy(gidx_hbm.at[wid], gidx_v)
    pltpu.sync_copy(sidx_hbm.at[wid], sidx_v)
    pltpu.sync_copy(deg_hbm.at[0], d0_v)
    pltpu.sync_copy(deg_hbm.at[1], d1_v)
    pltpu.sync_copy(s_hbm, val_v)

    @plsc.parallel_loop(0, NP // L, unroll=8)
    def valp(i):
        sl = pl.ds(i * L, L)
        d = d0_v[sl] + d1_v[sl]
        ii = lax.bitcast_convert_type(d, jnp.int32)
        y = lax.bitcast_convert_type(
            jnp.int32(0x5F3759DF) - lax.shift_right_arithmetic(ii, 1),
            jnp.float32)
        for _ in range(3):
            y = y * (1.5 - 0.5 * d * y * y)
        acc_v[sl] = jnp.zeros((L,), jnp.float32)
        val_v[sl] = jnp.where(d > 0, y, 0.0) * val_v[sl]

    @plsc.parallel_loop(0, EPW // L, unroll=8)
    def edge_body(i):
        gi = gidx_v[pl.ds(i * L, L)]
        si = sidx_v[pl.ds(i * L, L)]
        v = plsc.load_gather(val_v, [gi])
        plsc.addupdate_scatter(acc_v, [si], v)

    pltpu.sync_copy(acc_v, stage_sh.at[sid])
    plsc.subcore_barrier()
    pltpu.sync_copy(stage_sh.at[:, pl.ds(sid * NPW, NPW)], red_v)

    @plsc.parallel_loop(0, NPW // L, unroll=4)
    def red_body(i):
        sl = pl.ds(i * L, L)
        acc16 = red_v[0, sl]
        for r in range(1, NS):
            acc16 = acc16 + red_v[r, sl]
        out_v[sl] = acc16
    pltpu.sync_copy(out_v, out_hbm.at[cid, pl.ds(sid * NPW, NPW)])


@functools.partial(
    pl.kernel,
    out_type=jax.ShapeDtypeStruct((NW, LPW, NY2), jnp.float32),
    mesh=_mesh,
    compiler_params=pltpu.CompilerParams(needs_layout_passes=False),
    scratch_types=[
        pltpu.VMEM((LPW, NY2), jnp.float32),
        pltpu.VMEM((LPW, NY2), jnp.float32),
        pltpu.VMEM((NY2,), jnp.float32),
        pltpu.VMEM((L,), jnp.float32),
        pltpu.VMEM((CSZ,), jnp.int32),
        pltpu.VMEM((CSZ,), jnp.int32),
        pltpu.SemaphoreType.DMA,
        pltpu.SemaphoreType.DMA,
    ],
)
def _sc_vec_seg(pe_hbm, yt_hbm, w_hbm, sel_hbm, b_hbm, out_hbm,
                yv, accv, coef_v, bv, pe0, pe1, sem0, sem1):
    cid = lax.axis_index("c")
    sid = lax.axis_index("s")
    wid = sid * NC + cid
    pltpu.sync_copy(yt_hbm.at[wid], yv)
    pltpu.sync_copy(w_hbm.at[0], accv.at[0])
    pltpu.sync_copy(w_hbm.at[1], accv.at[1])
    pltpu.sync_copy(sel_hbm, accv.at[2])
    pltpu.sync_copy(b_hbm, bv)
    b16 = bv[...]

    @plsc.parallel_loop(0, NY2 // L, unroll=8)
    def cf(i):
        sl = pl.ds(i * L, L)
        z = accv[0, sl] + accv[1, sl] + b16
        w = 1.0 / (1.0 + jnp.exp(-z))
        coef_v[sl] = w * accv[2, sl]

    @plsc.parallel_loop(0, NY2 // L, unroll=8)
    def zb(i):
        sl = pl.ds(i * L, L)
        c16 = coef_v[sl]
        for l in range(LPW):
            accv[l, sl] = jnp.zeros((L,), jnp.float32)
            yv[l, sl] = yv[l, sl] * c16

    lane_idx = [jnp.full((L,), l, jnp.int32) for l in range(LPW)]

    def _process(buf):
        @plsc.parallel_loop(0, CSZ // L, unroll=8)
        def gb(g):
            pk = buf[pl.ds(g * L, L)]
            row16 = jnp.bitwise_and(pk, jnp.int32(16383))
            col16 = lax.shift_right_logical(pk, 14)
            for l in range(LPW):
                v = plsc.load_gather(yv, [lane_idx[l], row16])
                plsc.addupdate_scatter(accv, [lane_idx[l], col16], v)

    pltpu.sync_copy(pe_hbm.at[0], pe0)
    pltpu.async_copy(pe_hbm.at[1], pe1, sem1)

    def chunk_body(m, c):
        j = 2 * m
        more = m + 1 < NCH // 2

        @pl.when(m > 0)
        def _():
            pltpu.make_async_copy(pe_hbm.at[j], pe0, sem0).wait()
        _process(pe0)

        @pl.when(more)
        def _():
            pltpu.async_copy(pe_hbm.at[j + 2], pe0, sem0)
        pltpu.make_async_copy(pe_hbm.at[j + 1], pe1, sem1).wait()
        _process(pe1)

        @pl.when(more)
        def _():
            pltpu.async_copy(pe_hbm.at[j + 3], pe1, sem1)
        return c
    lax.fori_loop(0, NCH // 2, chunk_body, 0)

    pltpu.sync_copy(accv, out_hbm.at[wid])


def kernel(x, edge_index, W, b):
    row = edge_index[0]
    col = edge_index[1]
    pad = EP - E
    rowp = jnp.concatenate([row, jnp.full((pad,), N, jnp.int32)])
    colp = jnp.concatenate([col, jnp.full((pad,), N, jnp.int32)])
    rowf = rowp.reshape(NW, EPW)
    colf = colp.reshape(NW, EPW)
    packed = jnp.bitwise_or(rowp, colp << 14).reshape(NCH, CSZ)

    xt, s, a1, a2 = _tc1(x, W)

    ones_np = jnp.ones((NP,), jnp.float32)
    degp = _sc_scalar_seg(rowf, rowf, ones_np)

    sf = jnp.pad(s[:, 0], (0, NP - N))
    qp = _sc_score_seg(rowf, colf, degp, sf)
    sel, g = _tc3(qp[0, :N, None], qp[1, :N, None],
                  degp[0, :N, None], degp[1, :N, None], s, a1, a2)

    gf = jnp.pad(g[:, 0], (0, NP - N))
    wp = _sc_scalar_seg(rowf, colf, gf)
    selp = jnp.pad(sel[:, 0], (0, NY2 - N))
    bvec = jnp.full((L,), b[0], jnp.float32)

    xtpad = jnp.concatenate([xt, jnp.zeros((NY2 - N, D), jnp.float32)], axis=0)
    xt3 = xtpad.T.reshape(NW, LPW, NY2)
    at3 = _sc_vec_seg(packed, xt3, wp, selp, bvec)
    a_full = at3.reshape(D, NY2).T
    out = _tc5(xt, a_full[:N])
    return out
